# baseline (speedup 1.0000x reference)
import functools

import jax
import jax.numpy as jnp
from jax import lax
from jax.experimental import pallas as pl
from jax.experimental.pallas import tpu as pltpu
from jax.experimental.pallas import tpu_sc as plsc

NC = 2
NS = 16
NW = NC * NS
CH = 64

BM = 2048
L = 16
NSPLIT = 2
WW = 256
WO = 320


def _pack_pair(lo_f32, hi_f32):
    bl = lax.bitcast_convert_type(lo_f32, jnp.int32)
    bh = lax.bitcast_convert_type(hi_f32, jnp.int32)
    rl = ((bl + 0x7FFF + ((bl >> 16) & 1)) >> 16) & 0xFFFF
    rh = (bh + 0x7FFF + ((bh >> 16) & 1)) & jnp.int32(-65536)
    return rh | rl


def _precompute(eu_mlp, ei_mlp, eu_gmf, ei_gmf, w1a, w1b):
    def body(eum, eim, eug, eig, wa, wb, ut_o, it_o):
        pu = jnp.dot(eum[...], wa[...], preferred_element_type=jnp.float32)
        ut_o[:, :128] = _pack_pair(pu[:, :128], pu[:, 128:])
        ug = eug[...]
        nu = ug.shape[0]
        ut_o[:nu, 128:160] = _pack_pair(ug[:, :32], ug[:, 32:])
        ut_o[:, 160:] = jnp.zeros((eum.shape[0], WW - 160), jnp.int32)
        pi = jnp.dot(eim[...], wb[...], preferred_element_type=jnp.float32)
        it_o[:, :128] = _pack_pair(pi[:, :128], pi[:, 128:])
        ig = eig[...]
        it_o[:, 128:160] = _pack_pair(ig[:, :32], ig[:, 32:])
        it_o[:, 160:] = jnp.zeros((eim.shape[0], WW - 160), jnp.int32)

    return pl.pallas_call(
        body,
        out_shape=(
            jax.ShapeDtypeStruct((eu_mlp.shape[0], WW), jnp.int32),
            jax.ShapeDtypeStruct((ei_mlp.shape[0], WW), jnp.int32),
        ),
    )(eu_mlp, ei_mlp, eu_gmf, ei_gmf, w1a, w1b)


def _sc_fuse(uidx, sidx, utab, itab):
    B = uidx.shape[0]
    W = utab.shape[1]
    b_per_w = B // NW
    n_ch = b_per_w // CH
    mesh = plsc.VectorSubcoreMesh(core_axis_name="c", subcore_axis_name="s")

    @functools.partial(
        pl.kernel,
        mesh=mesh,
        out_type=jax.ShapeDtypeStruct((B, WO), jnp.float32),
        scratch_types=(
            pltpu.VMEM((b_per_w,), jnp.int32),
            pltpu.VMEM((b_per_w,), jnp.int32),
            pltpu.VMEM((CH, W), jnp.int32),
            pltpu.VMEM((CH, W), jnp.int32),
            pltpu.VMEM((CH, W), jnp.int32),
            pltpu.VMEM((CH, W), jnp.int32),
            pltpu.VMEM((CH, WO), jnp.float32),
            pltpu.VMEM((CH, WO), jnp.float32),
            pltpu.SemaphoreType.DMA,
            pltpu.SemaphoreType.DMA,
            pltpu.SemaphoreType.DMA,
            pltpu.SemaphoreType.DMA,
        ),
    )
    def fuse_k(uidx_h, sidx_h, ut_h, it_h, o_h,
               uidx_v, sidx_v, u0, i0, u1, i1, o0, o1, g0, g1, w0, w1):
        wid = lax.axis_index("s") * NC + lax.axis_index("c")
        base = wid * b_per_w
        cpu = pltpu.async_copy(uidx_h.at[pl.ds(base, b_per_w)], uidx_v, w0)
        cpi = pltpu.async_copy(sidx_h.at[pl.ds(base, b_per_w)], sidx_v, w0)
        cpu.wait()
        cpi.wait()

        ubuf = (u0, u1)
        ibuf = (i0, i1)
        obuf = (o0, o1)
        gsem = (g0, g1)
        wsem = (w0, w1)

        def fire(c):
            k = c % 2
            gu = pltpu.async_copy(
                ut_h.at[uidx_v.at[pl.ds(c * CH, CH)]], ubuf[k], gsem[k])
            gi = pltpu.async_copy(
                it_h.at[sidx_v.at[pl.ds(c * CH, CH)]], ibuf[k], gsem[k])
            return gu, gi

        def compute(c):
            k = c % 2
            u, i, o = ubuf[k], ibuf[k], obuf[k]

            @plsc.parallel_loop(0, CH, unroll=8)
            def row(r):
                def rd(ref, j):
                    w = ref[r, pl.ds(j * L, L)]
                    lo = lax.bitcast_convert_type(w << 16, jnp.float32)
                    hi = lax.bitcast_convert_type(w & jnp.int32(-65536),
                                                  jnp.float32)
                    return lo, hi

                for j in range(8):
                    ua, ub = rd(u, j)
                    ia, ib = rd(i, j)
                    o[r, pl.ds(L * j, L)] = ua + ia
                    o[r, pl.ds(128 + L * j, L)] = ub + ib
                for g in range(2):
                    ua, ub = rd(u, 8 + g)
                    ia, ib = rd(i, 8 + g)
                    o[r, pl.ds(256 + L * g, L)] = ua * ia
                    o[r, pl.ds(256 + 32 + L * g, L)] = ub * ib

        wb = [None, None]
        cur = fire(0)
        for c in range(n_ch):
            k = c % 2
            nxt = None
            if c + 1 < n_ch:
                if wb[(c + 1) % 2] is not None:
                    wb[(c + 1) % 2].wait()
                nxt = fire(c + 1)
            cur[0].wait()
            cur[1].wait()
            compute(c)
            wb[k] = pltpu.async_copy(
                obuf[k], o_h.at[pl.ds(base + c * CH, CH)], wsem[k])
            cur = nxt
        wb[0].wait()
        if wb[1] is not None:
            wb[1].wait()

    return fuse_k(uidx, sidx, utab, itab)


def _finish_body(hg_ref, b1, w2, b2, w3, b3, wpg, wph, bp, out):
    hg = hg_ref[...]
    h = jnp.maximum(hg[:, :256] + b1[...], 0.0)
    h = jnp.maximum(jnp.dot(h, w2[...], preferred_element_type=jnp.float32) + b2[...], 0.0)
    h = jnp.maximum(jnp.dot(h, w3[...], preferred_element_type=jnp.float32) + b3[...], 0.0)
    p = jnp.sum(hg[:, 256:320] * wpg[...], axis=1) + jnp.sum(h * wph[...], axis=1) + bp[0, 0]
    out[0, 0, :] = p


def _finish(hg, b1, W2, b2, W3, b3, wpg, wph, bp):
    B = hg.shape[0]
    H1, H2, H3 = 256, 128, 64
    nblk = B // BM
    full = lambda i: (0, 0)
    out = pl.pallas_call(
        _finish_body,
        grid=(nblk,),
        in_specs=[
            pl.BlockSpec((BM, WO), lambda i: (i, 0)),
            pl.BlockSpec((1, H1), full),
            pl.BlockSpec((H1, H2), full),
            pl.BlockSpec((1, H2), full),
            pl.BlockSpec((H2, H3), full),
            pl.BlockSpec((1, H3), full),
            pl.BlockSpec((1, 64), full),
            pl.BlockSpec((1, H3), full),
            pl.BlockSpec((1, 1), full),
        ],
        out_specs=pl.BlockSpec((1, 1, BM), lambda i: (i, 0, 0)),
        out_shape=jax.ShapeDtypeStruct((nblk, 1, BM), jnp.float32),
    )(hg, b1, W2, b2, W3, b3, wpg, wph, bp)
    return out.reshape(-1)


def kernel(userIdx, servIdx, eu_gmf, eu_mlp, ei_gmf, ei_mlp,
           W1, b1, W2, b2, W3, b3, Wp, bp):
    B = userIdx.shape[0]
    uidx = userIdx.astype(jnp.int32)
    sidx = servIdx.astype(jnp.int32)

    Dm = eu_mlp.shape[1]
    Dg = eu_gmf.shape[1]

    utab_i, itab_i = _precompute(eu_mlp, ei_mlp, eu_gmf, ei_gmf,
                                 W1[:Dm], W1[Dm:])

    b1r = b1.reshape(1, -1)
    b2r = b2.reshape(1, -1)
    b3r = b3.reshape(1, -1)
    wpg = Wp[:Dg].reshape(1, Dg)
    wph = Wp[Dg:].reshape(1, -1)
    bpr = bp.reshape(1, 1)

    H = B // NSPLIT
    parts = []
    for s in range(NSPLIT):
        hg = _sc_fuse(uidx[s * H:(s + 1) * H], sidx[s * H:(s + 1) * H],
                      utab_i, itab_i)
        parts.append(_finish(hg, b1r, W2, b2r, W3, b3r, wpg, wph, bpr))
    return jnp.concatenate(parts)


# speedup vs baseline: 1.0171x; 1.0171x over previous
import functools

import jax
import jax.numpy as jnp
from jax import lax
from jax.experimental import pallas as pl
from jax.experimental.pallas import tpu as pltpu
from jax.experimental.pallas import tpu_sc as plsc

NC = 2
NS = 16
NW = NC * NS
CH = 64

BM = 2048
L = 16
NSPLIT = 2
WW = 256
WO = 320


def _pack_pair(lo_f32, hi_f32):
    bl = lax.bitcast_convert_type(lo_f32, jnp.int32)
    bh = lax.bitcast_convert_type(hi_f32, jnp.int32)
    rl = ((bl + 0x7FFF + ((bl >> 16) & 1)) >> 16) & 0xFFFF
    rh = (bh + 0x7FFF + ((bh >> 16) & 1)) & jnp.int32(-65536)
    return rh | rl


def _precompute(eu_mlp, ei_mlp, eug_p, ei_gmf, w1a, w1b):
    def body(eum, eim, eug, eig, wa, wb, ut_o, it_o):
        pu = jnp.dot(eum[...], wa[...], preferred_element_type=jnp.float32)
        ut_o[:, :128] = _pack_pair(pu[:, :128], pu[:, 128:])
        ug = eug[...]
        ut_o[:, 128:160] = _pack_pair(ug[:, :32], ug[:, 32:])
        ut_o[:, 160:] = jnp.zeros((eum.shape[0], WW - 160), jnp.int32)
        pi = jnp.dot(eim[...], wb[...], preferred_element_type=jnp.float32)
        it_o[:, :128] = _pack_pair(pi[:, :128], pi[:, 128:])
        ig = eig[...]
        it_o[:, 128:160] = _pack_pair(ig[:, :32], ig[:, 32:])
        it_o[:, 160:] = jnp.zeros((eim.shape[0], WW - 160), jnp.int32)

    return pl.pallas_call(
        body,
        out_shape=(
            jax.ShapeDtypeStruct((eu_mlp.shape[0], WW), jnp.int32),
            jax.ShapeDtypeStruct((ei_mlp.shape[0], WW), jnp.int32),
        ),
    )(eu_mlp, ei_mlp, eug_p, ei_gmf, w1a, w1b)


def _sc_fuse(uidx, sidx, utab, itab):
    B = uidx.shape[0]
    W = utab.shape[1]
    b_per_w = B // NW
    n_ch = b_per_w // CH
    mesh = plsc.VectorSubcoreMesh(core_axis_name="c", subcore_axis_name="s")

    @functools.partial(
        pl.kernel,
        mesh=mesh,
        out_type=jax.ShapeDtypeStruct((B, WO), jnp.float32),
        scratch_types=(
            pltpu.VMEM((b_per_w,), jnp.int32),
            pltpu.VMEM((b_per_w,), jnp.int32),
            pltpu.VMEM((CH, W), jnp.int32),
            pltpu.VMEM((CH, W), jnp.int32),
            pltpu.VMEM((CH, W), jnp.int32),
            pltpu.VMEM((CH, W), jnp.int32),
            pltpu.VMEM((CH, WO), jnp.float32),
            pltpu.VMEM((CH, WO), jnp.float32),
            pltpu.SemaphoreType.DMA,
            pltpu.SemaphoreType.DMA,
            pltpu.SemaphoreType.DMA,
            pltpu.SemaphoreType.DMA,
        ),
    )
    def fuse_k(uidx_h, sidx_h, ut_h, it_h, o_h,
               uidx_v, sidx_v, u0, i0, u1, i1, o0, o1, g0, g1, w0, w1):
        wid = lax.axis_index("s") * NC + lax.axis_index("c")
        base = wid * b_per_w
        cpu = pltpu.async_copy(uidx_h.at[pl.ds(base, b_per_w)], uidx_v, w0)
        cpi = pltpu.async_copy(sidx_h.at[pl.ds(base, b_per_w)], sidx_v, w0)
        cpu.wait()
        cpi.wait()

        ubuf = (u0, u1)
        ibuf = (i0, i1)
        obuf = (o0, o1)
        gsem = (g0, g1)
        wsem = (w0, w1)

        def fire(c):
            k = c % 2
            gu = pltpu.async_copy(
                ut_h.at[uidx_v.at[pl.ds(c * CH, CH)]], ubuf[k], gsem[k])
            gi = pltpu.async_copy(
                it_h.at[sidx_v.at[pl.ds(c * CH, CH)]], ibuf[k], gsem[k])
            return gu, gi

        def compute(c):
            k = c % 2
            u, i, o = ubuf[k], ibuf[k], obuf[k]

            @plsc.parallel_loop(0, CH, unroll=8)
            def row(r):
                def rd(ref, j):
                    w = ref[r, pl.ds(j * L, L)]
                    lo = lax.bitcast_convert_type(w << 16, jnp.float32)
                    hi = lax.bitcast_convert_type(w & jnp.int32(-65536),
                                                  jnp.float32)
                    return lo, hi

                for j in range(8):
                    ua, ub = rd(u, j)
                    ia, ib = rd(i, j)
                    o[r, pl.ds(L * j, L)] = ua + ia
                    o[r, pl.ds(128 + L * j, L)] = ub + ib
                for g in range(2):
                    ua, ub = rd(u, 8 + g)
                    ia, ib = rd(i, 8 + g)
                    o[r, pl.ds(256 + L * g, L)] = ua * ia
                    o[r, pl.ds(256 + 32 + L * g, L)] = ub * ib

        wb = [None, None]
        cur = fire(0)
        for c in range(n_ch):
            k = c % 2
            nxt = None
            if c + 1 < n_ch:
                if wb[(c + 1) % 2] is not None:
                    wb[(c + 1) % 2].wait()
                nxt = fire(c + 1)
            cur[0].wait()
            cur[1].wait()
            compute(c)
            wb[k] = pltpu.async_copy(
                obuf[k], o_h.at[pl.ds(base + c * CH, CH)], wsem[k])
            cur = nxt
        wb[0].wait()
        if wb[1] is not None:
            wb[1].wait()

    return fuse_k(uidx, sidx, utab, itab)


def _finish_body(hg_ref, b1, w2, b2, w3, b3, wpg, wph, bp, out):
    hg = hg_ref[...]
    h = jnp.maximum(hg[:, :256] + b1[...], 0.0)
    h = jnp.maximum(jnp.dot(h, w2[...], preferred_element_type=jnp.float32) + b2[...], 0.0)
    h = jnp.maximum(jnp.dot(h, w3[...], preferred_element_type=jnp.float32) + b3[...], 0.0)
    p = jnp.sum(hg[:, 256:320] * wpg[...], axis=1) + jnp.sum(h * wph[...], axis=1) + bp[0, 0]
    out[0, 0, :] = p


def _finish(hg, b1, W2, b2, W3, b3, wpg, wph, bp):
    B = hg.shape[0]
    H1, H2, H3 = 256, 128, 64
    nblk = B // BM
    full = lambda i: (0, 0)
    out = pl.pallas_call(
        _finish_body,
        grid=(nblk,),
        in_specs=[
            pl.BlockSpec((BM, WO), lambda i: (i, 0)),
            pl.BlockSpec((1, H1), full),
            pl.BlockSpec((H1, H2), full),
            pl.BlockSpec((1, H2), full),
            pl.BlockSpec((H2, H3), full),
            pl.BlockSpec((1, H3), full),
            pl.BlockSpec((1, 64), full),
            pl.BlockSpec((1, H3), full),
            pl.BlockSpec((1, 1), full),
        ],
        out_specs=pl.BlockSpec((1, 1, BM), lambda i: (i, 0, 0)),
        out_shape=jax.ShapeDtypeStruct((nblk, 1, BM), jnp.float32),
    )(hg, b1, W2, b2, W3, b3, wpg, wph, bp)
    return out.reshape(-1)


def kernel(userIdx, servIdx, eu_gmf, eu_mlp, ei_gmf, ei_mlp,
           W1, b1, W2, b2, W3, b3, Wp, bp):
    B = userIdx.shape[0]
    uidx = userIdx.astype(jnp.int32)
    sidx = servIdx.astype(jnp.int32)

    Dm = eu_mlp.shape[1]
    Dg = eu_gmf.shape[1]

    eug_p = jnp.pad(eu_gmf, ((0, eu_mlp.shape[0] - eu_gmf.shape[0]), (0, 0)))
    utab_i, itab_i = _precompute(eu_mlp, ei_mlp, eug_p, ei_gmf,
                                 W1[:Dm], W1[Dm:])

    b1r = b1.reshape(1, -1)
    b2r = b2.reshape(1, -1)
    b3r = b3.reshape(1, -1)
    wpg = Wp[:Dg].reshape(1, Dg)
    wph = Wp[Dg:].reshape(1, -1)
    bpr = bp.reshape(1, 1)

    H = B // NSPLIT
    parts = []
    for s in range(NSPLIT):
        hg = _sc_fuse(uidx[s * H:(s + 1) * H], sidx[s * H:(s + 1) * H],
                      utab_i, itab_i)
        parts.append(_finish(hg, b1r, W2, b2r, W3, b3r, wpg, wph, bpr))
    return jnp.concatenate(parts)
